# baseline (speedup 1.0000x reference)
import functools

import jax
import jax.numpy as jnp
from jax import lax
from jax.experimental import pallas as pl
from jax.experimental.pallas import tpu as pltpu
from jax.experimental.pallas import tpu_sc as plsc

BATCH = 4096
SEQ = 50
EMBED = 128
NUM_CORES = 2
NUM_SUBCORES = 16
NW = NUM_CORES * NUM_SUBCORES
ROWS_W = BATCH // NW
CHB = 4
NCH = ROWS_W // CHB
NBUF = 4

_mesh = plsc.VectorSubcoreMesh(core_axis_name="c", subcore_axis_name="s",
                               num_cores=2)


@functools.partial(
    pl.kernel,
    mesh=_mesh,
    out_type=jax.ShapeDtypeStruct((BATCH, SEQ, EMBED), jnp.float32),
    scratch_types=[
        pltpu.VMEM((ROWS_W, SEQ), jnp.int32),
        pltpu.VMEM((NBUF, CHB, SEQ, EMBED), jnp.float32),
    ] + [pltpu.SemaphoreType.DMA] * (2 * NBUF),
)
def _sc_gather(table_hbm, idx_hbm, out_hbm, idx_v, rows_v, *sems):
    wid = lax.axis_index("s") * NUM_CORES + lax.axis_index("c")
    base = wid * ROWS_W
    gsem = sems[:NBUF]
    ssem = sems[NBUF:]
    pltpu.sync_copy(idx_hbm.at[pl.ds(base, ROWS_W)], idx_v)

    def start_gather(c, b):
        for g in range(CHB):
            pltpu.async_copy(table_hbm.at[idx_v.at[c * CHB + g]],
                             rows_v.at[b, g], gsem[b])

    def wait_gather(c, b):
        for g in range(CHB):
            pltpu.make_async_copy(table_hbm.at[idx_v.at[c * CHB + g]],
                                  rows_v.at[b, g], gsem[b]).wait()

    def start_store(c, b):
        pltpu.async_copy(rows_v.at[b],
                         out_hbm.at[pl.ds(base + c * CHB, CHB)], ssem[b])

    def wait_store(c, b):
        pltpu.make_async_copy(rows_v.at[b],
                              out_hbm.at[pl.ds(base + c * CHB, CHB)],
                              ssem[b]).wait()

    for c in range(NBUF - 1):
        start_gather(c, c)

    def body(i, carry):
        c0 = i * NBUF
        for k in range(NBUF):
            c = c0 + k
            ahead = (k + NBUF - 1) % NBUF

            @pl.when(c + NBUF - 1 < NCH)
            def _():
                @pl.when(c >= 1)
                def _():
                    wait_store(c - 1, ahead)

                start_gather(c + NBUF - 1, ahead)

            wait_gather(c, k)
            start_store(c, k)
        return carry

    lax.fori_loop(0, NCH // NBUF, body, 0)
    for c in range(NCH - NBUF, NCH):
        wait_store(c, c % NBUF)


def kernel(subword_sequences, token_embedding):
    return _sc_gather(token_embedding, subword_sequences.astype(jnp.int32))


# speedup vs baseline: 1.8082x; 1.8082x over previous
import functools

import jax
import jax.numpy as jnp
from jax import lax
from jax.experimental import pallas as pl
from jax.experimental.pallas import tpu as pltpu
from jax.experimental.pallas import tpu_sc as plsc

BATCH = 4096
SEQ = 50
EMBED = 128
NUM_CORES = 2
NUM_SUBCORES = 16
NW = NUM_CORES * NUM_SUBCORES
BW = BATCH // NW
NBUF = 5

_mesh = plsc.VectorSubcoreMesh(core_axis_name="c", subcore_axis_name="s",
                               num_cores=2)


@functools.partial(
    pl.kernel,
    mesh=_mesh,
    out_type=jax.ShapeDtypeStruct((SEQ, BATCH, EMBED), jnp.float32),
    scratch_types=[
        pltpu.VMEM((SEQ, BW), jnp.int32),
        pltpu.VMEM((NBUF, BW, EMBED), jnp.float32),
    ] + [pltpu.SemaphoreType.DMA] * (2 * NBUF),
)
def _sc_gather(table_hbm, idx_hbm, out_hbm, idx_v, rows_v, *sems):
    wid = lax.axis_index("s") * NUM_CORES + lax.axis_index("c")
    base = wid * BW
    gsem = sems[:NBUF]
    ssem = sems[NBUF:]
    pltpu.sync_copy(idx_hbm.at[:, pl.ds(base, BW)], idx_v)

    def start_gather(s, b):
        pltpu.async_copy(table_hbm.at[idx_v.at[s]], rows_v.at[b], gsem[b])

    def wait_gather(s, b):
        pltpu.make_async_copy(table_hbm.at[idx_v.at[s]], rows_v.at[b],
                              gsem[b]).wait()

    def start_store(s, b):
        pltpu.async_copy(rows_v.at[b], out_hbm.at[s, pl.ds(base, BW)],
                         ssem[b])

    def wait_store(s, b):
        pltpu.make_async_copy(rows_v.at[b], out_hbm.at[s, pl.ds(base, BW)],
                              ssem[b]).wait()

    for s in range(NBUF - 1):
        start_gather(s, s)

    def body(i, carry):
        s0 = i * NBUF
        for k in range(NBUF):
            s = s0 + k
            ahead = (k + NBUF - 1) % NBUF

            @pl.when(s + NBUF - 1 < SEQ)
            def _():
                @pl.when(s >= 1)
                def _():
                    wait_store(s - 1, ahead)

                start_gather(s + NBUF - 1, ahead)

            wait_gather(s, k)
            start_store(s, k)
        return carry

    lax.fori_loop(0, SEQ // NBUF, body, 0)
    for s in range(SEQ - NBUF, SEQ):
        wait_store(s, s % NBUF)


def kernel(subword_sequences, token_embedding):
    idx_t = subword_sequences.astype(jnp.int32).T
    out = _sc_gather(token_embedding, idx_t)
    return out.transpose(1, 0, 2)
